# baseline (speedup 1.0000x reference)
import functools

import jax
import jax.numpy as jnp
from jax import lax
from jax.experimental import pallas as pl
from jax.experimental.pallas import tpu as pltpu
from jax.experimental.pallas import tpu_sc as plsc

N_NODES = 10000
N_EDGES = 320000
EDIM = 16
NDIM_IN = 128
NDIM_OUT = 128

NWORK = 32
EPT = N_EDGES // NWORK
SUB = 80
CH = 2000
NCH = EPT // CH
RPC = CH // SUB
IPT = EPT // SUB
N_PAD = 10240
ROWS_PER_TILE = N_PAD // 16


MSG_BLK = 4000


def _msg_body(ef_ref, nwx_ref, out_ref):
    out_ref[...] = ef_ref[...] * nwx_ref[...]


def _make_msg(ef_r, nwx_r):
    grid = (N_EDGES // 8) // MSG_BLK
    return pl.pallas_call(
        _msg_body,
        grid=(grid,),
        in_specs=[
            pl.BlockSpec((MSG_BLK, 128), lambda i: (i, 0)),
            pl.BlockSpec((MSG_BLK, 128), lambda i: (i, 0)),
        ],
        out_specs=pl.BlockSpec((MSG_BLK, 128), lambda i: (i, 0)),
        out_shape=jax.ShapeDtypeStruct((N_EDGES // 8, 128), jnp.float32),
    )(ef_r, nwx_r)


def _sc_scatter_body(msg_hbm, dst_hbm, zeros_hbm, ones_hbm,
                     out_hbm, dout_hbm,
                     msg_v, dst_v, ones_v, acc_sh, dacc_sh):
    cid = lax.axis_index("c")
    sid = lax.axis_index("s")
    wid = cid * 16 + sid

    zbase = sid * ROWS_PER_TILE
    pltpu.sync_copy(zeros_hbm.at[pl.ds(zbase, ROWS_PER_TILE)],
                    acc_sh.at[pl.ds(zbase, ROWS_PER_TILE)])
    pltpu.sync_copy(zeros_hbm.at[pl.ds(zbase, ROWS_PER_TILE)],
                    dacc_sh.at[pl.ds(zbase, ROWS_PER_TILE)])
    pltpu.sync_copy(ones_hbm, ones_v)
    pltpu.sync_copy(dst_hbm.at[wid], dst_v)
    plsc.subcore_barrier()

    for ci in range(NCH):
        base = wid * EPT + ci * CH
        pltpu.sync_copy(msg_hbm.at[pl.ds(base, CH)], msg_v)
        for j in range(RPC):
            pltpu.sync_copy(msg_v.at[pl.ds(j * SUB, SUB)],
                            acc_sh.at[dst_v.at[ci * RPC + j]], add=True)
            pltpu.sync_copy(ones_v,
                            dacc_sh.at[dst_v.at[ci * RPC + j]], add=True)

    plsc.subcore_barrier()
    pltpu.sync_copy(acc_sh.at[pl.ds(zbase, ROWS_PER_TILE)],
                    out_hbm.at[cid, pl.ds(zbase, ROWS_PER_TILE)])
    pltpu.sync_copy(dacc_sh.at[pl.ds(zbase, ROWS_PER_TILE)],
                    dout_hbm.at[cid, pl.ds(zbase, ROWS_PER_TILE)])


_sc_scatter = functools.partial(
    pl.kernel,
    out_type=(jax.ShapeDtypeStruct((2, N_PAD, EDIM), jnp.float32),
              jax.ShapeDtypeStruct((2, N_PAD, EDIM), jnp.float32)),
    mesh=plsc.VectorSubcoreMesh(core_axis_name="c", subcore_axis_name="s"),
    compiler_params=pltpu.CompilerParams(use_tc_tiling_on_sc=False,
                                         needs_layout_passes=False),
    scratch_types=[
        pltpu.VMEM((CH, EDIM), jnp.float32),
        pltpu.VMEM((IPT, SUB), jnp.int32),
        pltpu.VMEM((SUB, EDIM), jnp.float32),
        pltpu.VMEM_SHARED((N_PAD, EDIM), jnp.float32),
        pltpu.VMEM_SHARED((N_PAD, EDIM), jnp.float32),
    ],
)(_sc_scatter_body)


def _final_body(parts_ref, degp_ref, nf_ref, wt_ref, b_ref, out_ref):
    s = parts_ref[0] + parts_ref[1]
    deg = degp_ref[0][:, 0:1] + degp_ref[1][:, 0:1]
    h_neigh = s[:N_NODES] / jnp.maximum(deg[:N_NODES], 1.0)
    h = jnp.concatenate([nf_ref[...], h_neigh], axis=1)
    acc = jnp.dot(h, wt_ref[...], preferred_element_type=jnp.float32)
    out_ref[...] = jnp.maximum(acc + b_ref[...], 0.0)


def _final(parts, degp, nf2, wt, b2):
    return pl.pallas_call(
        _final_body,
        out_shape=jax.ShapeDtypeStruct((N_NODES, NDIM_OUT), jnp.float32),
    )(parts, degp, nf2, wt, b2)


def kernel(nfeats, efeats, edge_index, norm_weight, W, b):
    ef_r = efeats.reshape(N_EDGES // 8, 128)
    nwx_r = jnp.broadcast_to(norm_weight[:, None],
                             (N_EDGES, EDIM)).reshape(N_EDGES // 8, 128)
    dst = edge_index[1].astype(jnp.int32).reshape(NWORK, IPT, SUB)
    zeros = jnp.zeros((N_PAD, EDIM), jnp.float32)
    onehot = jnp.tile(jnp.eye(1, EDIM, dtype=jnp.float32), (SUB, 1))
    wt = W.T
    b2 = b.reshape(1, NDIM_OUT)

    msg = _make_msg(ef_r, nwx_r).reshape(N_EDGES, EDIM)
    parts, degp = _sc_scatter(msg, dst, zeros, onehot)
    out2 = _final(parts, degp, nfeats.reshape(N_NODES, NDIM_IN), wt, b2)
    return out2.reshape(N_NODES, 1, NDIM_OUT)


# speedup vs baseline: 1.0670x; 1.0670x over previous
import functools

import jax
import jax.numpy as jnp
from jax import lax
from jax.experimental import pallas as pl
from jax.experimental.pallas import tpu as pltpu
from jax.experimental.pallas import tpu_sc as plsc

N_NODES = 10000
N_EDGES = 320000
EDIM = 16
NDIM_IN = 128
NDIM_OUT = 128

NWORK = 32
EPT = N_EDGES // NWORK
SUB = 80
CH = 2000
NCH = EPT // CH
RPC = CH // SUB
IPT = EPT // SUB
N_PAD = 10240
ROWS_PER_TILE = N_PAD // 16


MSG_BLK = 4000


def _msg_body(ef_ref, nwx_ref, out_ref):
    out_ref[...] = ef_ref[...] * nwx_ref[...]


def _make_msg(ef_r, nwx_r):
    grid = (N_EDGES // 8) // MSG_BLK
    return pl.pallas_call(
        _msg_body,
        grid=(grid,),
        in_specs=[
            pl.BlockSpec((MSG_BLK, 128), lambda i: (i, 0)),
            pl.BlockSpec((MSG_BLK, 128), lambda i: (i, 0)),
        ],
        out_specs=pl.BlockSpec((MSG_BLK, 128), lambda i: (i, 0)),
        out_shape=jax.ShapeDtypeStruct((N_EDGES // 8, 128), jnp.float32),
    )(ef_r, nwx_r)


def _sc_scatter_body(msg_hbm, dst_hbm, zeros_hbm, ones_hbm,
                     out_hbm, dout_hbm,
                     msg_a, msg_b, dst_v, ones_v, acc_sh, dacc_sh,
                     sem_in, sem_sc):
    cid = lax.axis_index("c")
    sid = lax.axis_index("s")
    wid = cid * 16 + sid

    zbase = sid * ROWS_PER_TILE
    pltpu.sync_copy(zeros_hbm.at[pl.ds(zbase, ROWS_PER_TILE)],
                    acc_sh.at[pl.ds(zbase, ROWS_PER_TILE)])
    pltpu.sync_copy(zeros_hbm.at[pl.ds(zbase, ROWS_PER_TILE)],
                    dacc_sh.at[pl.ds(zbase, ROWS_PER_TILE)])
    pltpu.sync_copy(ones_hbm, ones_v)
    pltpu.sync_copy(dst_hbm.at[wid], dst_v)
    plsc.subcore_barrier()

    bufs = [msg_a, msg_b]
    pending = [[], []]
    stage = [None, None]

    def start_stage(ci):
        base = wid * EPT + ci * CH
        return pltpu.async_copy(msg_hbm.at[pl.ds(base, CH)],
                                bufs[ci % 2], sem_in)

    stage[0] = start_stage(0)
    for ci in range(NCH):
        b = ci % 2
        nb = (ci + 1) % 2
        if ci + 1 < NCH:
            for h in pending[nb]:
                h.wait()
            pending[nb] = []
            stage[nb] = start_stage(ci + 1)
        stage[b].wait()
        hs = []
        for j in range(RPC):
            idx = dst_v.at[ci * RPC + j]
            hs.append(pltpu.async_copy(bufs[b].at[pl.ds(j * SUB, SUB)],
                                       acc_sh.at[idx], sem_sc, add=True))
            hs.append(pltpu.async_copy(ones_v,
                                       dacc_sh.at[idx], sem_sc, add=True))
        pending[b] = hs

    for bb in range(2):
        for h in pending[bb]:
            h.wait()

    plsc.subcore_barrier()
    pltpu.sync_copy(acc_sh.at[pl.ds(zbase, ROWS_PER_TILE)],
                    out_hbm.at[cid, pl.ds(zbase, ROWS_PER_TILE)])
    pltpu.sync_copy(dacc_sh.at[pl.ds(zbase, ROWS_PER_TILE)],
                    dout_hbm.at[cid, pl.ds(zbase, ROWS_PER_TILE)])


_sc_scatter = functools.partial(
    pl.kernel,
    out_type=(jax.ShapeDtypeStruct((2, N_PAD, EDIM), jnp.float32),
              jax.ShapeDtypeStruct((2, N_PAD, EDIM), jnp.float32)),
    mesh=plsc.VectorSubcoreMesh(core_axis_name="c", subcore_axis_name="s"),
    compiler_params=pltpu.CompilerParams(use_tc_tiling_on_sc=False,
                                         needs_layout_passes=False),
    scratch_types=[
        pltpu.VMEM((CH, EDIM), jnp.float32),
        pltpu.VMEM((CH, EDIM), jnp.float32),
        pltpu.VMEM((IPT, SUB), jnp.int32),
        pltpu.VMEM((SUB, EDIM), jnp.float32),
        pltpu.VMEM_SHARED((N_PAD, EDIM), jnp.float32),
        pltpu.VMEM_SHARED((N_PAD, EDIM), jnp.float32),
        pltpu.SemaphoreType.DMA,
        pltpu.SemaphoreType.DMA,
    ],
)(_sc_scatter_body)


def _final_body(parts_ref, degp_ref, nf_ref, wt_ref, b_ref, out_ref):
    s = parts_ref[0] + parts_ref[1]
    deg = degp_ref[0][:, 0:1] + degp_ref[1][:, 0:1]
    h_neigh = s[:N_NODES] / jnp.maximum(deg[:N_NODES], 1.0)
    h = jnp.concatenate([nf_ref[...], h_neigh], axis=1)
    acc = jnp.dot(h, wt_ref[...], preferred_element_type=jnp.float32)
    out_ref[...] = jnp.maximum(acc + b_ref[...], 0.0)


def _final(parts, degp, nf2, wt, b2):
    return pl.pallas_call(
        _final_body,
        out_shape=jax.ShapeDtypeStruct((N_NODES, NDIM_OUT), jnp.float32),
    )(parts, degp, nf2, wt, b2)


def kernel(nfeats, efeats, edge_index, norm_weight, W, b):
    ef_r = efeats.reshape(N_EDGES // 8, 128)
    nwx_r = jnp.broadcast_to(norm_weight[:, None],
                             (N_EDGES, EDIM)).reshape(N_EDGES // 8, 128)
    dst = edge_index[1].astype(jnp.int32).reshape(NWORK, IPT, SUB)
    zeros = jnp.zeros((N_PAD, EDIM), jnp.float32)
    onehot = jnp.tile(jnp.eye(1, EDIM, dtype=jnp.float32), (SUB, 1))
    wt = W.T
    b2 = b.reshape(1, NDIM_OUT)

    msg = _make_msg(ef_r, nwx_r).reshape(N_EDGES, EDIM)
    parts, degp = _sc_scatter(msg, dst, zeros, onehot)
    out2 = _final(parts, degp, nfeats.reshape(N_NODES, NDIM_IN), wt, b2)
    return out2.reshape(N_NODES, 1, NDIM_OUT)
